# baseline (speedup 1.0000x reference)
import functools

import jax
import jax.numpy as jnp
from jax import lax
from jax.experimental import pallas as pl
from jax.experimental.pallas import tpu as pltpu
from jax.experimental.pallas import tpu_sc as plsc

N_DST = 10000
D = 128
E_TOTAL = 320000

NUM_CORES = 2
NUM_SUBCORES = 16
NUM_WORKERS = NUM_CORES * NUM_SUBCORES

CHUNK = 128
PAGE = 40
NBUF = 2
CHUNKS_PER_WORKER = 80
E_PAD = NUM_WORKERS * CHUNKS_PER_WORKER * CHUNK

ACC_ROWS = 10112
ROWS_PER_TILE = ACC_ROWS // NUM_SUBCORES


def _segsum_body(src_rep_hbm, srcidx_hbm, dstidx_hbm, out_hbm,
                 srcidx_v, dstidx_v, buf0, buf1, zbuf, acc,
                 gsem0, gsem1, ssem0, ssem1):
    c = lax.axis_index("c")
    s = lax.axis_index("s")
    wid = c * NUM_SUBCORES + s

    zeros16 = jnp.zeros((16,), jnp.float32)
    for zr in range(8):
        for zc in range(D // 16):
            zbuf[zr, pl.ds(zc * 16, 16)] = zeros16

    r0 = s * ROWS_PER_TILE

    @pl.loop(0, ROWS_PER_TILE // 8)
    def _(k):
        pltpu.sync_copy(zbuf, acc.at[pl.ds(r0 + k * 8, 8)])
    plsc.subcore_barrier()

    bufs = (buf0, buf1)
    gsems = (gsem0, gsem1)
    ssems = (ssem0, ssem1)

    def start_gather(j, b):
        pltpu.async_copy(src_rep_hbm.at[srcidx_v.at[j]], bufs[b], gsems[b])

    def run_page(p):
        pltpu.sync_copy(srcidx_hbm.at[wid, pl.ds(p * PAGE, PAGE)], srcidx_v)
        pltpu.sync_copy(dstidx_hbm.at[wid, pl.ds(p * PAGE, PAGE)], dstidx_v)

        start_gather(0, 0)

        @pl.loop(0, PAGE, step=NBUF)
        def _(i):
            for b in range(NBUF):
                j = i + b
                pltpu.make_async_copy(src_rep_hbm.at[srcidx_v.at[j]],
                                      bufs[b], gsems[b]).wait()
                pltpu.async_copy(bufs[b], acc.at[dstidx_v.at[j]], ssems[b],
                                 add=True)
                nb = (b + 1) % NBUF

                @pl.when(j + 1 < PAGE)
                def _():
                    @pl.when(j >= 1)
                    def _():
                        pltpu.make_async_copy(bufs[nb],
                                              acc.at[dstidx_v.at[j - 1]],
                                              ssems[nb]).wait()

                    start_gather(j + 1, nb)

        for b in range(NBUF):
            j = PAGE - NBUF + b
            pltpu.make_async_copy(bufs[j % NBUF], acc.at[dstidx_v.at[j]],
                                  ssems[j % NBUF]).wait()

    for p in range(CHUNKS_PER_WORKER // PAGE):
        run_page(p)

    plsc.subcore_barrier()
    pltpu.sync_copy(acc.at[pl.ds(r0, ROWS_PER_TILE)],
                    out_hbm.at[c, pl.ds(r0, ROWS_PER_TILE)])


_segsum = functools.partial(
    pl.kernel,
    out_type=jax.ShapeDtypeStruct((NUM_CORES, ACC_ROWS, D), jnp.float32),
    mesh=plsc.VectorSubcoreMesh(core_axis_name="c", subcore_axis_name="s"),
    scratch_types=[
        pltpu.VMEM((PAGE, CHUNK), jnp.int32),
        pltpu.VMEM((PAGE, CHUNK), jnp.int32),
        pltpu.VMEM((CHUNK, D), jnp.float32),
        pltpu.VMEM((CHUNK, D), jnp.float32),
        pltpu.VMEM((8, D), jnp.float32),
        pltpu.VMEM_SHARED((ACC_ROWS, D), jnp.float32),
        pltpu.SemaphoreType.DMA,
        pltpu.SemaphoreType.DMA,
        pltpu.SemaphoreType.DMA,
        pltpu.SemaphoreType.DMA,
    ],
)(_segsum_body)


def _mlp_body(dst_ref, p_ref, w_ref, b_ref, o_ref):
    x1 = dst_ref[...]
    x2 = p_ref[0] + p_ref[1]
    w = w_ref[...]
    acc = lax.dot_general(x1, w[:, :D], (((1,), (1,)), ((), ())),
                          preferred_element_type=jnp.float32)
    acc = acc + lax.dot_general(x2, w[:, D:], (((1,), (1,)), ((), ())),
                                preferred_element_type=jnp.float32)
    o_ref[...] = jnp.maximum(acc + b_ref[...], 0.0)


def kernel(src_rep, dst_rep, edge_index, W, b):
    src = edge_index[0].astype(jnp.int32)
    dst = edge_index[1].astype(jnp.int32)
    e = src.shape[0]
    pad = E_PAD - e
    pad_src = jnp.arange(pad, dtype=jnp.int32) % src_rep.shape[0]
    pad_dst = N_DST + jnp.arange(pad, dtype=jnp.int32) % (ACC_ROWS - N_DST)
    src_p = jnp.concatenate([src, pad_src])
    dst_p = jnp.concatenate([dst, pad_dst.astype(jnp.int32)])

    src3 = src_p.reshape(NUM_WORKERS, CHUNKS_PER_WORKER, CHUNK)
    dst3 = dst_p.reshape(NUM_WORKERS, CHUNKS_PER_WORKER, CHUNK)

    partials = _segsum(src_rep, src3, dst3)

    n = dst_rep.shape[0]
    block = 1000
    grid = n // block
    out = pl.pallas_call(
        _mlp_body,
        grid=(grid,),
        in_specs=[
            pl.BlockSpec((block, D), lambda i: (i, 0)),
            pl.BlockSpec((NUM_CORES, block, D), lambda i: (0, i, 0)),
            pl.BlockSpec((D, 2 * D), lambda i: (0, 0)),
            pl.BlockSpec((1, D), lambda i: (0, 0)),
        ],
        out_specs=pl.BlockSpec((block, D), lambda i: (i, 0)),
        out_shape=jax.ShapeDtypeStruct((n, D), jnp.float32),
    )(dst_rep, partials, W, b.reshape(1, D))
    return out


# speedup vs baseline: 1.1390x; 1.1390x over previous
import functools

import jax
import jax.numpy as jnp
from jax import lax
from jax.experimental import pallas as pl
from jax.experimental.pallas import tpu as pltpu
from jax.experimental.pallas import tpu_sc as plsc

N_DST = 10000
D = 128
E_TOTAL = 320000

NUM_CORES = 2
NUM_SUBCORES = 16
NUM_WORKERS = NUM_CORES * NUM_SUBCORES

CHUNK = 128
PAGE = 40
NBUF = 2
CHUNKS_PER_WORKER = 80
E_PAD = NUM_WORKERS * CHUNKS_PER_WORKER * CHUNK

ACC_ROWS = 10112
ROWS_PER_TILE = ACC_ROWS // NUM_SUBCORES


def _segsum_body(src_rep_hbm, srcidx_hbm, dstidx_hbm, out_hbm,
                 srcidx_v, dstidx_v, buf0, buf1, zbuf, acc,
                 gsem0, gsem1, ssem0, ssem1):
    c = lax.axis_index("c")
    s = lax.axis_index("s")
    wid = c * NUM_SUBCORES + s

    zeros16 = jnp.zeros((16,), jnp.float32)
    for zr in range(8):
        for zc in range(D // 16):
            zbuf[zr, pl.ds(zc * 16, 16)] = zeros16

    r0 = s * ROWS_PER_TILE

    @pl.loop(0, ROWS_PER_TILE // 8)
    def _(k):
        pltpu.sync_copy(zbuf, acc.at[pl.ds(r0 + k * 8, 8)])
    plsc.subcore_barrier()

    bufs = (buf0, buf1)
    gsems = (gsem0, gsem1)
    ssems = (ssem0, ssem1)

    def start_gather(j, b):
        pltpu.async_copy(src_rep_hbm.at[srcidx_v.at[j]], bufs[b], gsems[b])

    def run_page(p):
        pltpu.sync_copy(srcidx_hbm.at[wid, pl.ds(p * PAGE, PAGE)], srcidx_v)
        pltpu.sync_copy(dstidx_hbm.at[wid, pl.ds(p * PAGE, PAGE)], dstidx_v)

        start_gather(0, 0)
        start_gather(1, 1)

        @pl.loop(0, PAGE, step=NBUF)
        def _(i):
            for b in range(NBUF):
                j = i + b
                pltpu.make_async_copy(src_rep_hbm.at[srcidx_v.at[j]],
                                      bufs[b], gsems[b]).wait()
                pltpu.async_copy(bufs[b], acc.at[dstidx_v.at[j]], ssems[b],
                                 add=True)
                pltpu.make_async_copy(bufs[b], acc.at[dstidx_v.at[j]],
                                      ssems[b]).wait()

                @pl.when(j + NBUF < PAGE)
                def _():
                    start_gather(j + NBUF, b)

    for p in range(CHUNKS_PER_WORKER // PAGE):
        run_page(p)

    plsc.subcore_barrier()
    pltpu.sync_copy(acc.at[pl.ds(r0, ROWS_PER_TILE)],
                    out_hbm.at[c, pl.ds(r0, ROWS_PER_TILE)])


_segsum = functools.partial(
    pl.kernel,
    out_type=jax.ShapeDtypeStruct((NUM_CORES, ACC_ROWS, D), jnp.float32),
    mesh=plsc.VectorSubcoreMesh(core_axis_name="c", subcore_axis_name="s"),
    scratch_types=[
        pltpu.VMEM((PAGE, CHUNK), jnp.int32),
        pltpu.VMEM((PAGE, CHUNK), jnp.int32),
        pltpu.VMEM((CHUNK, D), jnp.float32),
        pltpu.VMEM((CHUNK, D), jnp.float32),
        pltpu.VMEM((8, D), jnp.float32),
        pltpu.VMEM_SHARED((ACC_ROWS, D), jnp.float32),
        pltpu.SemaphoreType.DMA,
        pltpu.SemaphoreType.DMA,
        pltpu.SemaphoreType.DMA,
        pltpu.SemaphoreType.DMA,
    ],
)(_segsum_body)


def _mlp_body(dst_ref, p_ref, w_ref, b_ref, o_ref):
    x1 = dst_ref[...]
    x2 = p_ref[0] + p_ref[1]
    w = w_ref[...]
    acc = lax.dot_general(x1, w[:, :D], (((1,), (1,)), ((), ())),
                          preferred_element_type=jnp.float32)
    acc = acc + lax.dot_general(x2, w[:, D:], (((1,), (1,)), ((), ())),
                                preferred_element_type=jnp.float32)
    o_ref[...] = jnp.maximum(acc + b_ref[...], 0.0)


def kernel(src_rep, dst_rep, edge_index, W, b):
    src = edge_index[0].astype(jnp.int32)
    dst = edge_index[1].astype(jnp.int32)
    e = src.shape[0]
    pad = E_PAD - e
    pad_src = jnp.arange(pad, dtype=jnp.int32) % src_rep.shape[0]
    pad_dst = N_DST + jnp.arange(pad, dtype=jnp.int32) % (ACC_ROWS - N_DST)
    src_p = jnp.concatenate([src, pad_src])
    dst_p = jnp.concatenate([dst, pad_dst.astype(jnp.int32)])

    src3 = src_p.reshape(NUM_WORKERS, CHUNKS_PER_WORKER, CHUNK)
    dst3 = dst_p.reshape(NUM_WORKERS, CHUNKS_PER_WORKER, CHUNK)

    partials = _segsum(src_rep, src3, dst3)

    n = dst_rep.shape[0]
    block = 1000
    grid = n // block
    out = pl.pallas_call(
        _mlp_body,
        grid=(grid,),
        in_specs=[
            pl.BlockSpec((block, D), lambda i: (i, 0)),
            pl.BlockSpec((NUM_CORES, block, D), lambda i: (0, i, 0)),
            pl.BlockSpec((D, 2 * D), lambda i: (0, 0)),
            pl.BlockSpec((1, D), lambda i: (0, 0)),
        ],
        out_specs=pl.BlockSpec((block, D), lambda i: (i, 0)),
        out_shape=jax.ShapeDtypeStruct((n, D), jnp.float32),
    )(dst_rep, partials, W, b.reshape(1, D))
    return out
